# Initial kernel scaffold; baseline (speedup 1.0000x reference)
import jax
import jax.numpy as jnp
from jax.experimental import pallas as pl


def kernel(tokens, table):
    raise NotImplementedError("write your pallas kernel here")


# speedup vs baseline: 2.4229x; 2.4229x over previous
import functools
import math

import jax
import jax.numpy as jnp
from jax import lax
from jax.experimental import pallas as pl
from jax.experimental.pallas import tpu as pltpu
from jax.experimental.pallas import tpu_sc as plsc

_EMB = 128
_SCALE = math.sqrt(float(_EMB))
_NC = 2
_NS = 16
_NW = _NC * _NS
_K = 128
_LANES = 16


@functools.lru_cache(maxsize=None)
def _emb_call(nchunk):
    mesh = plsc.VectorSubcoreMesh(core_axis_name="c", subcore_axis_name="s")

    @functools.partial(
        pl.kernel,
        mesh=mesh,
        out_type=jax.ShapeDtypeStruct((_NW, nchunk, _K, _EMB), jnp.float32),
        scratch_types=[
            pltpu.VMEM((nchunk, _K), jnp.int32),
            pltpu.VMEM((_K, _EMB), jnp.float32),
            pltpu.SemaphoreType.DMA,
        ],
    )
    def body(tok_hbm, table_hbm, out_hbm, idx_v, rows_v, gsem):
        wid = lax.axis_index("s") * _NC + lax.axis_index("c")
        pltpu.sync_copy(tok_hbm.at[wid], idx_v)

        def chunk(j, carry):
            pltpu.async_copy(table_hbm.at[idx_v.at[j]], rows_v, gsem).wait()

            def scale_row(r, c2):
                for c in range(_EMB // _LANES):
                    sl = (r, pl.ds(c * _LANES, _LANES))
                    rows_v[sl] = rows_v[sl] * _SCALE
                return c2

            lax.fori_loop(0, _K, scale_row, 0)
            pltpu.sync_copy(rows_v, out_hbm.at[wid, j])
            return carry

        lax.fori_loop(0, nchunk, chunk, 0)

    return body


def kernel(tokens, table):
    b, s = tokens.shape
    rows = b * s
    assert rows % (_NW * _K) == 0
    nchunk = rows // (_NW * _K)
    tok = tokens.reshape(_NW, nchunk, _K).astype(jnp.int32)
    out = _emb_call(nchunk)(tok, table)
    return out.reshape(b, s, _EMB)
